# baseline (speedup 1.0000x reference)
import functools
import math

import jax
import jax.numpy as jnp
from jax import lax
from jax.experimental import pallas as pl
from jax.experimental.pallas import tpu as pltpu
from jax.experimental.pallas import tpu_sc as plsc

D_MODEL = 1024
SCALE = math.sqrt(D_MODEL)
LANES = 16
CHUNK = 16
NBUF = 4
N_WORKERS = 32
OUT_BYTES = CHUNK * D_MODEL * 4


@functools.partial(jax.jit, static_argnums=(2,))
def _embed_lookup(x, table, n_total):
    n_per_w = n_total // N_WORKERS
    n_chunks = n_per_w // CHUNK
    s = x.shape[1]
    w_per_row = s // n_per_w
    mesh = plsc.VectorSubcoreMesh(core_axis_name="c", subcore_axis_name="s")

    @functools.partial(
        pl.kernel,
        mesh=mesh,
        out_type=jax.ShapeDtypeStruct((n_total, D_MODEL), jnp.float32),
        scratch_types=[
            pltpu.VMEM((n_per_w + 2 * CHUNK,), jnp.int32),
            pltpu.VMEM((NBUF, CHUNK, D_MODEL), jnp.float32),
            pltpu.SemaphoreType.DMA((NBUF,)),
            pltpu.SemaphoreType.DMA((NBUF,)),
        ],
    )
    def k(x_hbm, table_hbm, out_hbm, idx_v, bufs, in_sem, out_sem):
        num_c = 2
        wid = lax.axis_index("s") * num_c + lax.axis_index("c")
        base = wid * n_per_w
        pltpu.sync_copy(
            x_hbm.at[wid // w_per_row, pl.ds((wid % w_per_row) * n_per_w, n_per_w)],
            idx_v.at[pl.ds(0, n_per_w)],
        )
        zeros16 = jnp.zeros((LANES,), jnp.int32)
        for t in range(2 * CHUNK // LANES):
            idx_v[pl.ds(n_per_w + t * LANES, LANES)] = zeros16

        def gather(c, b):
            pltpu.async_copy(
                table_hbm.at[idx_v.at[pl.ds(c * CHUNK, CHUNK)]],
                bufs.at[b],
                in_sem.at[b],
            )

        def wait_in(b):
            pltpu.make_async_copy(
                table_hbm.at[idx_v.at[pl.ds(0, CHUNK)]], bufs.at[b], in_sem.at[b]
            ).wait()

        def wait_out(b):
            pltpu.make_async_copy(
                bufs.at[b], out_hbm.at[pl.ds(base, CHUNK)], out_sem.at[b]
            ).wait()

        pltpu.async_copy(
            bufs.at[2],
            out_hbm.at[pl.ds(base + (n_chunks - 2) * CHUNK, CHUNK)],
            out_sem.at[2],
        )
        pltpu.async_copy(
            bufs.at[3],
            out_hbm.at[pl.ds(base + (n_chunks - 1) * CHUNK, CHUNK)],
            out_sem.at[3],
        )
        gather(0, 0)
        gather(1, 1)

        def group_body(i, carry):
            c0 = i * NBUF
            for b in range(NBUF):
                c = c0 + b
                bn = (b + 2) % NBUF
                wait_out(bn)
                gather(c + 2, bn)
                wait_in(b)

                @plsc.parallel_loop(0, CHUNK, unroll=1)
                def row_body(r):
                    for j in range(D_MODEL // LANES):
                        sl = pl.ds(j * LANES, LANES)
                        bufs[b, r, sl] = bufs[b, r, sl] * SCALE

                pltpu.async_copy(
                    bufs.at[b],
                    out_hbm.at[pl.ds(base + c * CHUNK, CHUNK)],
                    out_sem.at[b],
                )
            return carry

        lax.fori_loop(0, n_chunks // NBUF, group_body, 0)
        wait_in(0)
        wait_in(1)
        wait_out(2)
        wait_out(3)

    return k(x, table)


def kernel(x, table):
    b, s = x.shape
    n_total = b * s
    out = _embed_lookup(x, table, n_total)
    return out.reshape(b, s, D_MODEL)


# speedup vs baseline: 1.5579x; 1.5579x over previous
import functools
import math

import jax
import jax.numpy as jnp
from jax import lax
from jax.experimental import pallas as pl
from jax.experimental.pallas import tpu as pltpu
from jax.experimental.pallas import tpu_sc as plsc

D_MODEL = 1024
SCALE = math.sqrt(D_MODEL)
LANES = 16
CHUNK = 16
NBUF = 4
N_WORKERS = 32
OUT_BYTES = CHUNK * D_MODEL * 4


@functools.partial(jax.jit, static_argnums=(2,))
def _embed_lookup(x, table, n_total):
    n_per_w = n_total // N_WORKERS
    n_chunks = n_per_w // CHUNK
    s = x.shape[1]
    w_per_row = s // n_per_w
    mesh = plsc.VectorSubcoreMesh(core_axis_name="c", subcore_axis_name="s")

    @functools.partial(
        pl.kernel,
        mesh=mesh,
        out_type=jax.ShapeDtypeStruct((n_total, D_MODEL), jnp.float32),
        scratch_types=[
            pltpu.VMEM((n_per_w + 2 * CHUNK,), jnp.int32),
            pltpu.VMEM((NBUF, CHUNK, D_MODEL), jnp.float32),
            pltpu.SemaphoreType.DMA((NBUF,)),
            pltpu.SemaphoreType.DMA((NBUF,)),
        ],
    )
    def k(x_hbm, table_hbm, out_hbm, idx_v, bufs, in_sem, out_sem):
        num_c = 2
        wid = lax.axis_index("s") * num_c + lax.axis_index("c")
        base = wid * n_per_w
        pltpu.sync_copy(
            x_hbm.at[wid // w_per_row, pl.ds((wid % w_per_row) * n_per_w, n_per_w)],
            idx_v.at[pl.ds(0, n_per_w)],
        )
        for t in range(2 * CHUNK // LANES):
            idx_v[pl.ds(n_per_w + t * LANES, LANES)] = lax.iota(
                jnp.int32, LANES
            ) + (wid * 2 * CHUNK + t * LANES)

        def gather(c, b):
            pltpu.async_copy(
                table_hbm.at[idx_v.at[pl.ds(c * CHUNK, CHUNK)]],
                bufs.at[b],
                in_sem.at[b],
            )

        def wait_in(b):
            pltpu.make_async_copy(
                table_hbm.at[idx_v.at[pl.ds(0, CHUNK)]], bufs.at[b], in_sem.at[b]
            ).wait()

        def wait_out(b):
            pltpu.make_async_copy(
                bufs.at[b], out_hbm.at[pl.ds(base, CHUNK)], out_sem.at[b]
            ).wait()

        pltpu.async_copy(
            bufs.at[2],
            out_hbm.at[pl.ds(base + (n_chunks - 2) * CHUNK, CHUNK)],
            out_sem.at[2],
        )
        pltpu.async_copy(
            bufs.at[3],
            out_hbm.at[pl.ds(base + (n_chunks - 1) * CHUNK, CHUNK)],
            out_sem.at[3],
        )
        gather(0, 0)
        gather(1, 1)

        def group_body(i, carry):
            c0 = i * NBUF
            for b in range(NBUF):
                c = c0 + b
                bn = (b + 2) % NBUF
                wait_out(bn)
                gather(c + 2, bn)
                wait_in(b)

                @plsc.parallel_loop(0, CHUNK, unroll=1)
                def row_body(r):
                    for j in range(D_MODEL // LANES):
                        sl = pl.ds(j * LANES, LANES)
                        bufs[b, r, sl] = bufs[b, r, sl] * SCALE

                pltpu.async_copy(
                    bufs.at[b],
                    out_hbm.at[pl.ds(base + c * CHUNK, CHUNK)],
                    out_sem.at[b],
                )
            return carry

        lax.fori_loop(0, n_chunks // NBUF, group_body, 0)
        wait_in(0)
        wait_in(1)
        wait_out(2)
        wait_out(3)

    return k(x, table)


def kernel(x, table):
    b, s = x.shape
    n_total = b * s
    out = _embed_lookup(x, table, n_total)
    return out.reshape(b, s, D_MODEL)
